# initial kernel scaffold (unmeasured)
import jax
import jax.numpy as jnp
from jax import lax
from jax.experimental import pallas as pl
from jax.experimental.pallas import tpu as pltpu

N_DEV = 8


def kernel(x, w_mat, scale_x, scale_w):
    m, k_loc = x.shape
    k_loc2, n = w_mat.shape
    assert k_loc == k_loc2
    m_per = m // N_DEV

    def body(x_ref, w_ref, sx_ref, sw_ref, out_ref,
             comm_ref, send_sems, recv_sems, ack_sem):
        my = lax.axis_index("i")
        right = lax.rem(my + 1, N_DEV)
        left = lax.rem(my + N_DEV - 1, N_DEV)

        w_bf = w_ref[...].astype(jnp.bfloat16)

        def partial(c):
            xs = x_ref[pl.ds(c * m_per, m_per), :].astype(jnp.bfloat16)
            return lax.dot_general(
                xs, w_bf, (((1,), (0,)), ((), ())),
                preferred_element_type=jnp.float32,
            )

        comm_ref[0] = partial(left).astype(jnp.bfloat16)

        rdmas = []
        for s in range(N_DEV - 1):
            rdma = pltpu.make_async_remote_copy(
                src_ref=comm_ref.at[s],
                dst_ref=comm_ref.at[s + 1],
                send_sem=send_sems.at[s],
                recv_sem=recv_sems.at[s],
                device_id=(right,),
                device_id_type=pl.DeviceIdType.MESH,
            )
            rdma.start()
            rdmas.append(rdma)
            c_recv = lax.rem(my + 2 * N_DEV - 2 - s, N_DEV)
            part = partial(c_recv)
            rdma.wait_recv()
            if s < N_DEV - 2:
                comm_ref[s + 1] = (
                    comm_ref[s + 1].astype(jnp.float32) + part
                ).astype(jnp.bfloat16)
            else:
                total = comm_ref[s + 1].astype(jnp.float32) + part
                scale = sx_ref[0] * sw_ref[0]
                out_ref[...] = jnp.maximum(total * scale, 0.0)

        for rdma in rdmas:
            rdma.wait_send()

        pl.semaphore_signal(
            ack_sem, inc=1,
            device_id=(left,), device_id_type=pl.DeviceIdType.MESH,
        )
        pl.semaphore_wait(ack_sem, 1)

    return pl.pallas_call(
        body,
        out_shape=jax.ShapeDtypeStruct((m_per, n), jnp.float32),
        in_specs=[
            pl.BlockSpec(memory_space=pltpu.VMEM),
            pl.BlockSpec(memory_space=pltpu.VMEM),
            pl.BlockSpec(memory_space=pltpu.SMEM),
            pl.BlockSpec(memory_space=pltpu.SMEM),
        ],
        out_specs=pl.BlockSpec(memory_space=pltpu.VMEM),
        scratch_shapes=[
            pltpu.VMEM((N_DEV, m_per, n), jnp.bfloat16),
            pltpu.SemaphoreType.DMA((N_DEV - 1,)),
            pltpu.SemaphoreType.DMA((N_DEV - 1,)),
            pltpu.SemaphoreType.REGULAR,
        ],
        compiler_params=pltpu.CompilerParams(collective_id=0),
    )(x, w_mat, scale_x, scale_w)


# baseline (device time: 192646 ns/iter reference)
import jax
import jax.numpy as jnp
from jax import lax
from jax.experimental import pallas as pl
from jax.experimental.pallas import tpu as pltpu

N_DEV = 8


def kernel(x, w_mat, scale_x, scale_w):
    m, k_loc = x.shape
    k_loc2, n = w_mat.shape
    assert k_loc == k_loc2
    m_per = m // N_DEV

    def body(x_ref, w_ref, sx_ref, sw_ref, out_ref,
             comm_ref, send_sems, recv_sems, ack_sem):
        my = lax.axis_index("i")
        right = lax.rem(my + 1, N_DEV)
        left = lax.rem(my + N_DEV - 1, N_DEV)

        w_bf = w_ref[...].astype(jnp.bfloat16)

        def partial(c):
            xs = x_ref[pl.ds(c * m_per, m_per), :].astype(jnp.bfloat16)
            return lax.dot_general(
                xs, w_bf, (((1,), (0,)), ((), ())),
                preferred_element_type=jnp.float32,
            )

        comm_ref[0] = partial(left).astype(jnp.bfloat16)

        rdmas = []
        for s in range(N_DEV - 1):
            rdma = pltpu.make_async_remote_copy(
                src_ref=comm_ref.at[s],
                dst_ref=comm_ref.at[s + 1],
                send_sem=send_sems.at[s],
                recv_sem=recv_sems.at[s],
                device_id=(right,),
                device_id_type=pl.DeviceIdType.MESH,
            )
            rdma.start()
            rdmas.append(rdma)
            c_recv = lax.rem(my + 2 * N_DEV - 2 - s, N_DEV)
            part = partial(c_recv)
            rdma.wait_recv()
            if s < N_DEV - 2:
                comm_ref[s + 1] = (
                    comm_ref[s + 1].astype(jnp.float32) + part
                ).astype(jnp.bfloat16)
            else:
                total = comm_ref[s + 1].astype(jnp.float32) + part
                scale = sx_ref[0] * sw_ref[0]
                out_ref[...] = jnp.maximum(total * scale, 0.0)

        for rdma in rdmas:
            rdma.wait_send()

        pl.semaphore_signal(
            ack_sem, inc=1,
            device_id=(left,), device_id_type=pl.DeviceIdType.MESH,
        )
        pl.semaphore_wait(ack_sem, 1)

    return pl.pallas_call(
        body,
        out_shape=jax.ShapeDtypeStruct((m_per, n), jnp.float32),
        in_specs=[
            pl.BlockSpec(memory_space=pltpu.VMEM),
            pl.BlockSpec(memory_space=pltpu.VMEM),
            pl.BlockSpec(memory_space=pltpu.SMEM),
            pl.BlockSpec(memory_space=pltpu.SMEM),
        ],
        out_specs=pl.BlockSpec(memory_space=pltpu.VMEM),
        scratch_shapes=[
            pltpu.VMEM((N_DEV, m_per, n), jnp.bfloat16),
            pltpu.SemaphoreType.DMA((N_DEV - 1,)),
            pltpu.SemaphoreType.DMA((N_DEV - 1,)),
            pltpu.SemaphoreType.REGULAR,
        ],
    )(x, w_mat, scale_x, scale_w)


# device time: 114937 ns/iter; 1.6761x vs baseline; 1.6761x over previous
import jax
import jax.numpy as jnp
from jax import lax
from jax.experimental import pallas as pl
from jax.experimental.pallas import tpu as pltpu

N_DEV = 8


def kernel(x, w_mat, scale_x, scale_w):
    m, k_loc = x.shape
    k_loc2, n = w_mat.shape
    assert k_loc == k_loc2
    m_per = m // N_DEV
    n_half = n // 2

    def body(x_ref, w_ref, sx_ref, sw_ref, out_ref,
             comm_r, comm_l, send_r, recv_r, send_l, recv_l, ack_sem):
        my = lax.axis_index("i")
        right = lax.rem(my + 1, N_DEV)
        left = lax.rem(my + N_DEV - 1, N_DEV)

        w_bf = w_ref[...].astype(jnp.bfloat16)

        def partial(c, lo):
            xs = x_ref[pl.ds(c * m_per, m_per), :].astype(jnp.bfloat16)
            return lax.dot_general(
                xs, w_bf[:, lo:lo + n_half], (((1,), (0,)), ((), ())),
                preferred_element_type=jnp.float32,
            )

        comm_r[0] = partial(left, 0).astype(jnp.bfloat16)
        comm_l[0] = partial(right, n_half).astype(jnp.bfloat16)

        rdmas = []
        for s in range(N_DEV - 1):
            rdma_r = pltpu.make_async_remote_copy(
                src_ref=comm_r.at[s],
                dst_ref=comm_r.at[s + 1],
                send_sem=send_r.at[s],
                recv_sem=recv_r.at[s],
                device_id=(right,),
                device_id_type=pl.DeviceIdType.MESH,
            )
            rdma_l = pltpu.make_async_remote_copy(
                src_ref=comm_l.at[s],
                dst_ref=comm_l.at[s + 1],
                send_sem=send_l.at[s],
                recv_sem=recv_l.at[s],
                device_id=(left,),
                device_id_type=pl.DeviceIdType.MESH,
            )
            rdma_r.start()
            rdma_l.start()
            rdmas += [rdma_r, rdma_l]
            c_r = lax.rem(my + 2 * N_DEV - 2 - s, N_DEV)
            c_l = lax.rem(my + 2 + s, N_DEV)
            part_r = partial(c_r, 0)
            part_l = partial(c_l, n_half)
            rdma_r.wait_recv()
            if s < N_DEV - 2:
                comm_r[s + 1] = (
                    comm_r[s + 1].astype(jnp.float32) + part_r
                ).astype(jnp.bfloat16)
            rdma_l.wait_recv()
            if s < N_DEV - 2:
                comm_l[s + 1] = (
                    comm_l[s + 1].astype(jnp.float32) + part_l
                ).astype(jnp.bfloat16)
            else:
                scale = sx_ref[0] * sw_ref[0]
                tot_r = comm_r[s + 1].astype(jnp.float32) + part_r
                out_ref[:, 0:n_half] = jnp.maximum(tot_r * scale, 0.0)
                tot_l = comm_l[s + 1].astype(jnp.float32) + part_l
                out_ref[:, n_half:n] = jnp.maximum(tot_l * scale, 0.0)

        for rdma in rdmas:
            rdma.wait_send()

        for nbr in (left, right):
            pl.semaphore_signal(
                ack_sem, inc=1,
                device_id=(nbr,), device_id_type=pl.DeviceIdType.MESH,
            )
        pl.semaphore_wait(ack_sem, 2)

    return pl.pallas_call(
        body,
        out_shape=jax.ShapeDtypeStruct((m_per, n), jnp.float32),
        in_specs=[
            pl.BlockSpec(memory_space=pltpu.VMEM),
            pl.BlockSpec(memory_space=pltpu.VMEM),
            pl.BlockSpec(memory_space=pltpu.SMEM),
            pl.BlockSpec(memory_space=pltpu.SMEM),
        ],
        out_specs=pl.BlockSpec(memory_space=pltpu.VMEM),
        scratch_shapes=[
            pltpu.VMEM((N_DEV, m_per, n_half), jnp.bfloat16),
            pltpu.VMEM((N_DEV, m_per, n_half), jnp.bfloat16),
            pltpu.SemaphoreType.DMA((N_DEV - 1,)),
            pltpu.SemaphoreType.DMA((N_DEV - 1,)),
            pltpu.SemaphoreType.DMA((N_DEV - 1,)),
            pltpu.SemaphoreType.DMA((N_DEV - 1,)),
            pltpu.SemaphoreType.REGULAR,
        ],
    )(x, w_mat, scale_x, scale_w)


# device time: 98187 ns/iter; 1.9620x vs baseline; 1.1706x over previous
import jax
import jax.numpy as jnp
from jax import lax
from jax.experimental import pallas as pl
from jax.experimental.pallas import tpu as pltpu

N_DEV = 8
N_SUB = 2


def kernel(x, w_mat, scale_x, scale_w):
    m, k_loc = x.shape
    k_loc2, n = w_mat.shape
    assert k_loc == k_loc2
    m_per = m // N_DEV
    n_half = n // 2
    n_sub = n_half // N_SUB

    def body(x_ref, w_ref, sx_ref, sw_ref, out_ref,
             comm_r, comm_l, send_r, recv_r, send_l, recv_l, ack_sem):
        my = lax.axis_index("i")
        right = lax.rem(my + 1, N_DEV)
        left = lax.rem(my + N_DEV - 1, N_DEV)

        w_bf = w_ref[...].astype(jnp.bfloat16)

        def partial(c, col0):
            xs = x_ref[pl.ds(c * m_per, m_per), :].astype(jnp.bfloat16)
            return lax.dot_general(
                xs, w_bf[:, col0:col0 + n_sub], (((1,), (0,)), ((), ())),
                preferred_element_type=jnp.float32,
            )

        def make_rdma(comm, send, recv, s, q, dst):
            return pltpu.make_async_remote_copy(
                src_ref=comm.at[s, q],
                dst_ref=comm.at[s + 1, q],
                send_sem=send.at[s, q],
                recv_sem=recv.at[s, q],
                device_id=(dst,),
                device_id_type=pl.DeviceIdType.MESH,
            )

        col_r = lambda q: q * n_sub
        col_l = lambda q: n_half + q * n_sub

        rdmas = []

        for q in range(N_SUB):
            comm_r[0, q] = partial(left, col_r(q)).astype(jnp.bfloat16)
            rr = make_rdma(comm_r, send_r, recv_r, 0, q, right)
            rr.start()
            comm_l[0, q] = partial(right, col_l(q)).astype(jnp.bfloat16)
            rl = make_rdma(comm_l, send_l, recv_l, 0, q, left)
            rl.start()
            rdmas += [rr, rl]

        for s in range(N_DEV - 1):
            c_r = lax.rem(my + 2 * N_DEV - 2 - s, N_DEV)
            c_l = lax.rem(my + 2 + s, N_DEV)
            last = s == N_DEV - 2
            parts = [(partial(c_r, col_r(q)), partial(c_l, col_l(q)))
                     for q in range(N_SUB)]
            cur = [(make_rdma(comm_r, send_r, recv_r, s, q, right),
                    make_rdma(comm_l, send_l, recv_l, s, q, left))
                   for q in range(N_SUB)]
            scale = sx_ref[0] * sw_ref[0]
            for q in range(N_SUB):
                rr, rl = cur[q]
                pr, plft = parts[q]
                rr.wait_recv()
                tot = comm_r[s + 1, q].astype(jnp.float32) + pr
                if not last:
                    comm_r[s + 1, q] = tot.astype(jnp.bfloat16)
                    nxt = make_rdma(comm_r, send_r, recv_r, s + 1, q, right)
                    nxt.start()
                    rdmas.append(nxt)
                else:
                    out_ref[:, pl.ds(col_r(q), n_sub)] = (
                        jnp.maximum(tot * scale, 0.0))
                rl.wait_recv()
                tot = comm_l[s + 1, q].astype(jnp.float32) + plft
                if not last:
                    comm_l[s + 1, q] = tot.astype(jnp.bfloat16)
                    nxt = make_rdma(comm_l, send_l, recv_l, s + 1, q, left)
                    nxt.start()
                    rdmas.append(nxt)
                else:
                    out_ref[:, pl.ds(col_l(q), n_sub)] = (
                        jnp.maximum(tot * scale, 0.0))

        for rdma in rdmas:
            rdma.wait_send()

        for nbr in (left, right):
            pl.semaphore_signal(
                ack_sem, inc=1,
                device_id=(nbr,), device_id_type=pl.DeviceIdType.MESH,
            )
        pl.semaphore_wait(ack_sem, 2)

    return pl.pallas_call(
        body,
        out_shape=jax.ShapeDtypeStruct((m_per, n), jnp.float32),
        in_specs=[
            pl.BlockSpec(memory_space=pltpu.VMEM),
            pl.BlockSpec(memory_space=pltpu.VMEM),
            pl.BlockSpec(memory_space=pltpu.SMEM),
            pl.BlockSpec(memory_space=pltpu.SMEM),
        ],
        out_specs=pl.BlockSpec(memory_space=pltpu.VMEM),
        scratch_shapes=[
            pltpu.VMEM((N_DEV, N_SUB, m_per, n_sub), jnp.bfloat16),
            pltpu.VMEM((N_DEV, N_SUB, m_per, n_sub), jnp.bfloat16),
            pltpu.SemaphoreType.DMA((N_DEV - 1, N_SUB)),
            pltpu.SemaphoreType.DMA((N_DEV - 1, N_SUB)),
            pltpu.SemaphoreType.DMA((N_DEV - 1, N_SUB)),
            pltpu.SemaphoreType.DMA((N_DEV - 1, N_SUB)),
            pltpu.SemaphoreType.REGULAR,
        ],
    )(x, w_mat, scale_x, scale_w)
